# initial kernel scaffold (unmeasured)
import numpy as np
import jax
import jax.numpy as jnp
from jax import lax
from jax.experimental import pallas as pl
from jax.experimental.pallas import tpu as pltpu

N_DEV = 8
B, SQ, D = 16, 512, 1024
B_LOC = B // N_DEV
HQ_LOC, DH = 8, 128
SCALE = 0.08838834764831843

MESH = pl.DeviceIdType.MESH


def _rope_tables():
    inv = 1.0 / (10000.0 ** (np.arange(0, DH, 2) / DH))
    pos = np.arange(SQ)[:, None] * inv[None, :]
    cos = np.repeat(np.cos(pos), 2, axis=-1)
    sin = np.repeat(np.sin(pos), 2, axis=-1)
    cos_t = np.tile(cos, (1, HQ_LOC)).astype(np.float32)
    sin_t = np.tile(sin, (1, HQ_LOC)).astype(np.float32)
    even = (np.arange(D) % 2 == 0)[None, :]
    sin_a = np.where(even, -sin_t, 0.0).astype(np.float32)
    sin_b = np.where(~even, sin_t, 0.0).astype(np.float32)
    return cos_t, sin_a, sin_b


_COS_T, _SIN_A, _SIN_B = _rope_tables()


def _neighbor_barrier(my):
    left = lax.rem(my + N_DEV - 1, N_DEV)
    right = lax.rem(my + 1, N_DEV)
    sem = pltpu.get_barrier_semaphore()
    for nbr in (left, right):
        pl.semaphore_signal(sem, inc=1, device_id=(nbr,), device_id_type=MESH)
    pl.semaphore_wait(sem, 2)


def _ag_body(x_ref, out_ref, send_sems, recv_sems):
    my = lax.axis_index("i")
    right = lax.rem(my + 1, N_DEV)
    _neighbor_barrier(my)

    out_ref[pl.ds(my * B_LOC, B_LOC)] = x_ref[...]

    for h in range(N_DEV - 1):
        origin = lax.rem(my - h + N_DEV, N_DEV)
        src = x_ref if h == 0 else out_ref.at[pl.ds(origin * B_LOC, B_LOC)]
        rdma = pltpu.make_async_remote_copy(
            src_ref=src,
            dst_ref=out_ref.at[pl.ds(origin * B_LOC, B_LOC)],
            send_sem=send_sems.at[h],
            recv_sem=recv_sems.at[h],
            device_id=(right,),
            device_id_type=MESH,
        )
        rdma.start()
        rdma.wait()


def _all_gather_x(x):
    return pl.pallas_call(
        _ag_body,
        out_shape=jax.ShapeDtypeStruct((B, SQ, D), jnp.float32),
        in_specs=[pl.BlockSpec(memory_space=pltpu.VMEM)],
        out_specs=pl.BlockSpec(memory_space=pltpu.VMEM),
        scratch_shapes=[
            pltpu.SemaphoreType.DMA((N_DEV - 1,)),
            pltpu.SemaphoreType.DMA((N_DEV - 1,)),
        ],
        compiler_params=pltpu.CompilerParams(
            collective_id=0, vmem_limit_bytes=100 * 1024 * 1024
        ),
    )(x)


def _attn_body(x_ref, wq_ref, wk_ref, wv_ref, wo_ref, cos_ref, sa_ref,
               sb_ref, out_ref, ctx_ref):
    x = x_ref[0]
    cos = cos_ref[...]
    sa = sa_ref[...]
    sb = sb_ref[...]

    def rope(t):
        return (t * cos
                + pltpu.roll(t, -1, 1) * sa
                + pltpu.roll(t, 1, 1) * sb)

    q = rope(jnp.dot(x, wq_ref[...], preferred_element_type=jnp.float32))
    k = rope(jnp.dot(x, wk_ref[...], preferred_element_type=jnp.float32))
    v = jnp.dot(x, wv_ref[...], preferred_element_type=jnp.float32)

    for h in range(HQ_LOC):
        sl = slice(h * DH, (h + 1) * DH)
        s = lax.dot_general(
            q[:, sl], k[:, sl], (((1,), (1,)), ((), ())),
            preferred_element_type=jnp.float32,
        ) * SCALE
        s = s - jnp.max(s, axis=1, keepdims=True)
        e = jnp.exp(s)
        w = e / jnp.sum(e, axis=1, keepdims=True)
        ctx_ref[:, sl] = jnp.dot(w, v[:, sl],
                                 preferred_element_type=jnp.float32)

    out_ref[0] = jnp.dot(ctx_ref[...], wo_ref[...],
                         preferred_element_type=jnp.float32)


def _attn_partial(x_full, Wq, Wk, Wv, Wo):
    w_spec = pl.BlockSpec((D, D), lambda b: (0, 0))
    t_spec = pl.BlockSpec((SQ, D), lambda b: (0, 0))
    return pl.pallas_call(
        _attn_body,
        grid=(B,),
        out_shape=jax.ShapeDtypeStruct((B, SQ, D), jnp.float32),
        in_specs=[
            pl.BlockSpec((1, SQ, D), lambda b: (b, 0, 0)),
            w_spec, w_spec, w_spec, w_spec,
            t_spec, t_spec, t_spec,
        ],
        out_specs=pl.BlockSpec((1, SQ, D), lambda b: (b, 0, 0)),
        scratch_shapes=[pltpu.VMEM((SQ, D), jnp.float32)],
        compiler_params=pltpu.CompilerParams(
            dimension_semantics=("arbitrary",),
            vmem_limit_bytes=100 * 1024 * 1024,
        ),
    )(x_full, Wq, Wk, Wv, Wo,
      jnp.asarray(_COS_T), jnp.asarray(_SIN_A), jnp.asarray(_SIN_B))


def _rs_body(p_ref, out_ref, send_buf, chunk_buf, recv_bufs, send_sems,
             recv_sems, copy_sem):
    my = lax.axis_index("i")
    right = lax.rem(my + 1, N_DEV)
    _neighbor_barrier(my)

    c0 = lax.rem(my + N_DEV - 1, N_DEV)
    cp = pltpu.make_async_copy(
        p_ref.at[pl.ds(c0 * B_LOC, B_LOC)], send_buf, copy_sem)
    cp.start()
    cp.wait()

    for s in range(N_DEV - 1):
        rdma = pltpu.make_async_remote_copy(
            src_ref=send_buf,
            dst_ref=recv_bufs.at[s],
            send_sem=send_sems.at[s],
            recv_sem=recv_sems.at[s],
            device_id=(right,),
            device_id_type=MESH,
        )
        rdma.start()
        c = lax.rem(my - s - 2 + 2 * N_DEV, N_DEV)
        cp = pltpu.make_async_copy(
            p_ref.at[pl.ds(c * B_LOC, B_LOC)], chunk_buf, copy_sem)
        cp.start()
        cp.wait()
        rdma.wait()
        if s < N_DEV - 2:
            send_buf[...] = recv_bufs[s] + chunk_buf[...]
        else:
            out_ref[...] = recv_bufs[s] + chunk_buf[...]


def _reduce_scatter(p):
    return pl.pallas_call(
        _rs_body,
        out_shape=jax.ShapeDtypeStruct((B_LOC, SQ, D), jnp.float32),
        in_specs=[pl.BlockSpec(memory_space=pltpu.MemorySpace.ANY)],
        out_specs=pl.BlockSpec(memory_space=pltpu.VMEM),
        scratch_shapes=[
            pltpu.VMEM((B_LOC, SQ, D), jnp.float32),
            pltpu.VMEM((B_LOC, SQ, D), jnp.float32),
            pltpu.VMEM((N_DEV - 1, B_LOC, SQ, D), jnp.float32),
            pltpu.SemaphoreType.DMA((N_DEV - 1,)),
            pltpu.SemaphoreType.DMA((N_DEV - 1,)),
            pltpu.SemaphoreType.DMA,
        ],
        compiler_params=pltpu.CompilerParams(
            collective_id=1, vmem_limit_bytes=100 * 1024 * 1024
        ),
    )(p)


def kernel(x, Wq, Wk, Wv, Wo):
    x_full = _all_gather_x(x)
    partial = _attn_partial(x_full, Wq, Wk, Wv, Wo)
    return _reduce_scatter(partial)


# baseline (device time: 897799 ns/iter reference)
import numpy as np
import jax
import jax.numpy as jnp
from jax import lax
from jax.experimental import pallas as pl
from jax.experimental.pallas import tpu as pltpu

N_DEV = 8
B, SQ, D = 16, 512, 1024
B_LOC = B // N_DEV
HQ_LOC, DH = 8, 128
SCALE = 0.08838834764831843

MESH = pl.DeviceIdType.MESH


def _rope_tables():
    inv = 1.0 / (10000.0 ** (np.arange(0, DH, 2) / DH))
    pos = np.arange(SQ)[:, None] * inv[None, :]
    cos = np.repeat(np.cos(pos), 2, axis=-1)
    sin = np.repeat(np.sin(pos), 2, axis=-1)
    cos_t = np.tile(cos, (1, HQ_LOC)).astype(np.float32)
    sin_t = np.tile(sin, (1, HQ_LOC)).astype(np.float32)
    even = (np.arange(D) % 2 == 0)[None, :]
    sin_a = np.where(even, -sin_t, 0.0).astype(np.float32)
    sin_b = np.where(~even, sin_t, 0.0).astype(np.float32)
    return cos_t, sin_a, sin_b


_COS_T, _SIN_A, _SIN_B = _rope_tables()


def _neighbor_barrier(my):
    left = lax.rem(my + N_DEV - 1, N_DEV)
    right = lax.rem(my + 1, N_DEV)
    sem = pltpu.get_barrier_semaphore()
    for nbr in (left, right):
        pl.semaphore_signal(sem, inc=1, device_id=(nbr,), device_id_type=MESH)
    pl.semaphore_wait(sem, 2)


def _ag_body(x_ref, out_ref, send_sems, recv_sems):
    my = lax.axis_index("i")
    right = lax.rem(my + 1, N_DEV)
    _neighbor_barrier(my)

    out_ref[pl.ds(my * B_LOC, B_LOC)] = x_ref[...]

    for h in range(N_DEV - 1):
        origin = lax.rem(my - h + N_DEV, N_DEV)
        src = x_ref if h == 0 else out_ref.at[pl.ds(origin * B_LOC, B_LOC)]
        rdma = pltpu.make_async_remote_copy(
            src_ref=src,
            dst_ref=out_ref.at[pl.ds(origin * B_LOC, B_LOC)],
            send_sem=send_sems.at[h],
            recv_sem=recv_sems.at[h],
            device_id=(right,),
            device_id_type=MESH,
        )
        rdma.start()
        rdma.wait()


def _all_gather_x(x):
    return pl.pallas_call(
        _ag_body,
        out_shape=jax.ShapeDtypeStruct((B, SQ, D), jnp.float32),
        in_specs=[pl.BlockSpec(memory_space=pltpu.VMEM)],
        out_specs=pl.BlockSpec(memory_space=pltpu.VMEM),
        scratch_shapes=[
            pltpu.SemaphoreType.DMA((N_DEV - 1,)),
            pltpu.SemaphoreType.DMA((N_DEV - 1,)),
        ],
        compiler_params=pltpu.CompilerParams(
            collective_id=0, vmem_limit_bytes=100 * 1024 * 1024
        ),
    )(x)


def _attn_body(x_ref, wq_ref, wk_ref, wv_ref, wo_ref, cos_ref, sa_ref,
               sb_ref, out_ref, ctx_ref):
    x = x_ref[0]
    cos = cos_ref[...]
    sa = sa_ref[...]
    sb = sb_ref[...]

    def rope(t):
        return (t * cos
                + pltpu.roll(t, D - 1, 1) * sa
                + pltpu.roll(t, 1, 1) * sb)

    q = rope(jnp.dot(x, wq_ref[...], preferred_element_type=jnp.float32))
    k = rope(jnp.dot(x, wk_ref[...], preferred_element_type=jnp.float32))
    v = jnp.dot(x, wv_ref[...], preferred_element_type=jnp.float32)

    for h in range(HQ_LOC):
        sl = slice(h * DH, (h + 1) * DH)
        s = lax.dot_general(
            q[:, sl], k[:, sl], (((1,), (1,)), ((), ())),
            preferred_element_type=jnp.float32,
        ) * SCALE
        s = s - jnp.max(s, axis=1, keepdims=True)
        e = jnp.exp(s)
        w = e / jnp.sum(e, axis=1, keepdims=True)
        ctx_ref[:, sl] = jnp.dot(w, v[:, sl],
                                 preferred_element_type=jnp.float32)

    out_ref[0] = jnp.dot(ctx_ref[...], wo_ref[...],
                         preferred_element_type=jnp.float32)


def _attn_partial(x_full, Wq, Wk, Wv, Wo):
    w_spec = pl.BlockSpec((D, D), lambda b: (0, 0))
    t_spec = pl.BlockSpec((SQ, D), lambda b: (0, 0))
    return pl.pallas_call(
        _attn_body,
        grid=(B,),
        out_shape=jax.ShapeDtypeStruct((B, SQ, D), jnp.float32),
        in_specs=[
            pl.BlockSpec((1, SQ, D), lambda b: (b, 0, 0)),
            w_spec, w_spec, w_spec, w_spec,
            t_spec, t_spec, t_spec,
        ],
        out_specs=pl.BlockSpec((1, SQ, D), lambda b: (b, 0, 0)),
        scratch_shapes=[pltpu.VMEM((SQ, D), jnp.float32)],
        compiler_params=pltpu.CompilerParams(
            dimension_semantics=("arbitrary",),
            vmem_limit_bytes=100 * 1024 * 1024,
        ),
    )(x_full, Wq, Wk, Wv, Wo,
      jnp.asarray(_COS_T), jnp.asarray(_SIN_A), jnp.asarray(_SIN_B))


def _rs_body(p_ref, out_ref, send_buf, chunk_buf, recv_bufs, send_sems,
             recv_sems, copy_sem):
    my = lax.axis_index("i")
    right = lax.rem(my + 1, N_DEV)
    _neighbor_barrier(my)

    c0 = lax.rem(my + N_DEV - 1, N_DEV)
    cp = pltpu.make_async_copy(
        p_ref.at[pl.ds(c0 * B_LOC, B_LOC)], send_buf, copy_sem)
    cp.start()
    cp.wait()

    for s in range(N_DEV - 1):
        rdma = pltpu.make_async_remote_copy(
            src_ref=send_buf,
            dst_ref=recv_bufs.at[s],
            send_sem=send_sems.at[s],
            recv_sem=recv_sems.at[s],
            device_id=(right,),
            device_id_type=MESH,
        )
        rdma.start()
        c = lax.rem(my - s - 2 + 2 * N_DEV, N_DEV)
        cp = pltpu.make_async_copy(
            p_ref.at[pl.ds(c * B_LOC, B_LOC)], chunk_buf, copy_sem)
        cp.start()
        cp.wait()
        rdma.wait()
        if s < N_DEV - 2:
            send_buf[...] = recv_bufs[s] + chunk_buf[...]
        else:
            out_ref[...] = recv_bufs[s] + chunk_buf[...]


def _reduce_scatter(p):
    return pl.pallas_call(
        _rs_body,
        out_shape=jax.ShapeDtypeStruct((B_LOC, SQ, D), jnp.float32),
        in_specs=[pl.BlockSpec(memory_space=pltpu.MemorySpace.HBM)],
        out_specs=pl.BlockSpec(memory_space=pltpu.VMEM),
        scratch_shapes=[
            pltpu.VMEM((B_LOC, SQ, D), jnp.float32),
            pltpu.VMEM((B_LOC, SQ, D), jnp.float32),
            pltpu.VMEM((N_DEV - 1, B_LOC, SQ, D), jnp.float32),
            pltpu.SemaphoreType.DMA((N_DEV - 1,)),
            pltpu.SemaphoreType.DMA((N_DEV - 1,)),
            pltpu.SemaphoreType.DMA,
        ],
        compiler_params=pltpu.CompilerParams(
            collective_id=1, vmem_limit_bytes=100 * 1024 * 1024
        ),
    )(p)


def kernel(x, Wq, Wk, Wv, Wo):
    x_full = _all_gather_x(x)
    partial = _attn_partial(x_full, Wq, Wk, Wv, Wo)
    return _reduce_scatter(partial)


# device time: 386659 ns/iter; 2.3219x vs baseline; 2.3219x over previous
import numpy as np
import jax
import jax.numpy as jnp
from jax import lax
from jax.experimental import pallas as pl
from jax.experimental.pallas import tpu as pltpu

N_DEV = 8
B, SQ, D = 16, 512, 1024
B_LOC = B // N_DEV
HQ_LOC, DH = 8, 128
SCALE = 0.08838834764831843

MESH = pl.DeviceIdType.MESH


def _rope_tables():
    inv = 1.0 / (10000.0 ** (np.arange(0, DH, 2) / DH))
    pos = np.arange(SQ)[:, None] * inv[None, :]
    cos = np.repeat(np.cos(pos), 2, axis=-1).astype(np.float32)
    sin = np.repeat(np.sin(pos), 2, axis=-1).astype(np.float32)
    even = (np.arange(DH) % 2 == 0)[None, :]
    sin_a = np.where(even, -sin, 0.0).astype(np.float32)
    sin_b = np.where(~even, sin, 0.0).astype(np.float32)
    return cos, sin_a, sin_b


_COS_T, _SIN_A, _SIN_B = _rope_tables()


def _body(x_ref, wq_ref, wk_ref, wv_ref, wo_ref, cos_ref, sa_ref, sb_ref,
          out_ref, xa_buf, xb_buf, aa_buf, ab_buf, p_ref,
          xa_s, xa_r, xb_s, xb_r, aa_s, aa_r, ab_s, ab_r,
          xa_c, xb_c, aa_c, ab_c):
    my = lax.axis_index("i")
    left = lax.rem(my + N_DEV - 1, N_DEV)
    right = lax.rem(my + 1, N_DEV)

    sem = pltpu.get_barrier_semaphore()
    for nbr in (left, right):
        pl.semaphore_signal(sem, inc=1, device_id=(nbr,), device_id_type=MESH)
    pl.semaphore_wait(sem, 2)

    cos = cos_ref[...]
    sa = sa_ref[...]
    sb = sb_ref[...]

    def rope(t):
        return (t * cos
                + pltpu.roll(t, DH - 1, 1) * sa
                + pltpu.roll(t, 1, 1) * sb)

    def compute_partial(x):
        for h in range(HQ_LOC):
            sl = slice(h * DH, (h + 1) * DH)
            qh = rope(jnp.dot(x, wq_ref[:, sl],
                              preferred_element_type=jnp.float32))
            kh = rope(jnp.dot(x, wk_ref[:, sl],
                              preferred_element_type=jnp.float32))
            vh = jnp.dot(x, wv_ref[:, sl],
                         preferred_element_type=jnp.float32)
            s_ = lax.dot_general(
                qh, kh, (((1,), (1,)), ((), ())),
                preferred_element_type=jnp.float32) * SCALE
            s_ = s_ - jnp.max(s_, axis=1, keepdims=True)
            e = jnp.exp(s_)
            w = e / jnp.sum(e, axis=1, keepdims=True)
            ch = jnp.dot(w, vh, preferred_element_type=jnp.float32)
            contrib = jnp.dot(ch, wo_ref[sl, :],
                              preferred_element_type=jnp.float32)
            if h == 0:
                p_ref[...] = contrib
            else:
                p_ref[...] = p_ref[...] + contrib

    def x_desc(j, xbuf, xin, ssem, rsem, dst_dev):
        return pltpu.make_async_remote_copy(
            src_ref=xin if j == 0 else xbuf.at[j % 2],
            dst_ref=xbuf.at[(j + 1) % 2],
            send_sem=ssem.at[j], recv_sem=rsem.at[j],
            device_id=(dst_dev,), device_id_type=MESH)

    def a_desc(j, abuf, out_slot, ssem, rsem, dst_dev):
        return pltpu.make_async_remote_copy(
            src_ref=abuf.at[j % 2],
            dst_ref=out_ref.at[out_slot] if j == 7 else abuf.at[(j + 1) % 2],
            send_sem=ssem.at[j], recv_sem=rsem.at[j],
            device_id=(dst_dev,), device_id_type=MESH)

    xa = lambda j: x_desc(j, xa_buf, x_ref.at[0], xa_s, xa_r, left)
    aa = lambda j: a_desc(j, aa_buf, 0, aa_s, aa_r, left)
    xb = lambda j: x_desc(j, xb_buf, x_ref.at[1], xb_s, xb_r, right)
    ab = lambda j: a_desc(j, ab_buf, 1, ab_s, ab_r, right)

    def credit(sem, dev):
        pl.semaphore_signal(sem, inc=1, device_id=(dev,),
                            device_id_type=MESH)

    xa(0).start()
    xb(0).start()

    for s in range(N_DEV):
        for (x_in, xbuf, abuf, xd, ad, xc, ac, up) in (
            (x_ref, xa_buf, aa_buf, xa, aa, xa_c, aa_c, right),
            (x_ref, xb_buf, ab_buf, xb, ab, xb_c, ab_c, left),
        ):
            half = 0 if xbuf is xa_buf else 1
            x_cur = x_in[half] if s == 0 else xbuf[s % 2]
            compute_partial(x_cur)

            if s <= 6:
                xd(s).wait_send()
                if 1 <= s <= 5:
                    credit(xc, up)
                xd(s).wait_recv()
                if s <= 5:
                    if s >= 1:
                        pl.semaphore_wait(xc, 1)
                    xd(s + 1).start()

            if s == 0:
                abuf[0] = p_ref[...]
            else:
                ad(s - 1).wait_send()
                if s <= 6:
                    credit(ac, up)
                ad(s - 1).wait_recv()
                abuf[s % 2] = abuf[s % 2] + p_ref[...]
                if s >= 2:
                    pl.semaphore_wait(ac, 1)
            ad(s).start()

    aa(7).wait_recv()
    ab(7).wait_recv()
    aa(7).wait_send()
    ab(7).wait_send()


def kernel(x, Wq, Wk, Wv, Wo):
    vm = pl.BlockSpec(memory_space=pltpu.VMEM)
    n_x, n_a = N_DEV - 1, N_DEV
    return pl.pallas_call(
        _body,
        out_shape=jax.ShapeDtypeStruct((B_LOC, SQ, D), jnp.float32),
        in_specs=[vm] * 8,
        out_specs=vm,
        scratch_shapes=[
            pltpu.VMEM((2, SQ, D), jnp.float32),
            pltpu.VMEM((2, SQ, D), jnp.float32),
            pltpu.VMEM((2, SQ, D), jnp.float32),
            pltpu.VMEM((2, SQ, D), jnp.float32),
            pltpu.VMEM((SQ, D), jnp.float32),
            pltpu.SemaphoreType.DMA((n_x,)),
            pltpu.SemaphoreType.DMA((n_x,)),
            pltpu.SemaphoreType.DMA((n_x,)),
            pltpu.SemaphoreType.DMA((n_x,)),
            pltpu.SemaphoreType.DMA((n_a,)),
            pltpu.SemaphoreType.DMA((n_a,)),
            pltpu.SemaphoreType.DMA((n_a,)),
            pltpu.SemaphoreType.DMA((n_a,)),
            pltpu.SemaphoreType.REGULAR,
            pltpu.SemaphoreType.REGULAR,
            pltpu.SemaphoreType.REGULAR,
            pltpu.SemaphoreType.REGULAR,
        ],
        compiler_params=pltpu.CompilerParams(
            collective_id=0, vmem_limit_bytes=62 * 1024 * 1024
        ),
    )(x, Wq, Wk, Wv, Wo,
      jnp.asarray(_COS_T), jnp.asarray(_SIN_A), jnp.asarray(_SIN_B))


# device time: 344346 ns/iter; 2.6073x vs baseline; 1.1229x over previous
import numpy as np
import jax
import jax.numpy as jnp
from jax import lax
from jax.experimental import pallas as pl
from jax.experimental.pallas import tpu as pltpu

N_DEV = 8
B, SQ, D = 16, 512, 1024
B_LOC = B // N_DEV
HQ_LOC, DH = 8, 128
SCALE = 0.08838834764831843

MESH = pl.DeviceIdType.MESH


def _rope_tables():
    inv = 1.0 / (10000.0 ** (np.arange(0, DH, 2) / DH))
    pos = np.arange(SQ)[:, None] * inv[None, :]
    cos = np.repeat(np.cos(pos), 2, axis=-1).astype(np.float32)
    sin = np.repeat(np.sin(pos), 2, axis=-1).astype(np.float32)
    even = (np.arange(DH) % 2 == 0)[None, :]
    sin_a = np.where(even, -sin, 0.0).astype(np.float32)
    sin_b = np.where(~even, sin, 0.0).astype(np.float32)
    return cos, sin_a, sin_b


_COS_T, _SIN_A, _SIN_B = _rope_tables()


def _body(x_ref, wq_ref, wk_ref, wv_ref, wo_ref, cos_ref, sa_ref, sb_ref,
          out_ref, xa_buf, xb_buf, aa_buf, ab_buf, p_ref, xsend,
          xa_s, xa_r, xb_s, xb_r, aa_s, aa_r, ab_s, ab_r,
          xa_c, xb_c, aa_c, ab_c):
    my = lax.axis_index("i")
    left = lax.rem(my + N_DEV - 1, N_DEV)
    right = lax.rem(my + 1, N_DEV)

    sem = pltpu.get_barrier_semaphore()
    for nbr in (left, right):
        pl.semaphore_signal(sem, inc=1, device_id=(nbr,), device_id_type=MESH)
    pl.semaphore_wait(sem, 2)

    cos = cos_ref[...]
    sa = sa_ref[...]
    sb = sb_ref[...]

    def rope(t):
        return (t * cos
                + pltpu.roll(t, DH - 1, 1) * sa
                + pltpu.roll(t, 1, 1) * sb)

    def compute_partial(x):
        p_ref[...] = jnp.zeros((SQ, D), jnp.float32)

        def head(h, _):
            qh = rope(jnp.dot(x, wq_ref[h],
                              preferred_element_type=jnp.float32))
            kh = rope(jnp.dot(x, wk_ref[h],
                              preferred_element_type=jnp.float32))
            vh = jnp.dot(x, wv_ref[h],
                         preferred_element_type=jnp.float32)
            s_ = lax.dot_general(
                qh, kh, (((1,), (1,)), ((), ())),
                preferred_element_type=jnp.float32) * SCALE
            s_ = s_ - jnp.max(s_, axis=1, keepdims=True)
            e = jnp.exp(s_)
            w = e / jnp.sum(e, axis=1, keepdims=True)
            ch = jnp.dot(w, vh, preferred_element_type=jnp.float32)
            p_ref[...] = p_ref[...] + jnp.dot(
                ch, wo_ref[h], preferred_element_type=jnp.float32)
            return 0

        lax.fori_loop(0, HQ_LOC, head, 0)

    def x_desc(j, xbuf, xin, ssem, rsem, dst_dev):
        return pltpu.make_async_remote_copy(
            src_ref=xin if j == 0 else xbuf.at[j % 2],
            dst_ref=xbuf.at[(j + 1) % 2],
            send_sem=ssem.at[j], recv_sem=rsem.at[j],
            device_id=(dst_dev,), device_id_type=MESH)

    def a_desc(j, abuf, out_slot, ssem, rsem, dst_dev):
        return pltpu.make_async_remote_copy(
            src_ref=abuf.at[j % 2],
            dst_ref=out_ref.at[out_slot] if j == 7 else abuf.at[(j + 1) % 2],
            send_sem=ssem.at[j], recv_sem=rsem.at[j],
            device_id=(dst_dev,), device_id_type=MESH)

    xa = lambda j: x_desc(j, xa_buf, xsend.at[0], xa_s, xa_r, left)
    aa = lambda j: a_desc(j, aa_buf, 0, aa_s, aa_r, left)
    xb = lambda j: x_desc(j, xb_buf, xsend.at[1], xb_s, xb_r, right)
    ab = lambda j: a_desc(j, ab_buf, 1, ab_s, ab_r, right)

    def credit(sem, dev):
        pl.semaphore_signal(sem, inc=1, device_id=(dev,),
                            device_id_type=MESH)

    xsend[0] = x_ref[0].astype(jnp.bfloat16)
    xsend[1] = x_ref[1].astype(jnp.bfloat16)
    xa(0).start()
    xb(0).start()

    for s in range(N_DEV):
        for (x_in, xbuf, abuf, xd, ad, xc, ac, up) in (
            (x_ref, xa_buf, aa_buf, xa, aa, xa_c, aa_c, right),
            (x_ref, xb_buf, ab_buf, xb, ab, xb_c, ab_c, left),
        ):
            half = 0 if xbuf is xa_buf else 1
            x_cur = (x_in[half] if s == 0
                     else xbuf[s % 2].astype(jnp.float32))
            compute_partial(x_cur)

            if s <= 6:
                xd(s).wait_send()
                if 1 <= s <= 5:
                    credit(xc, up)
                xd(s).wait_recv()
                if s <= 5:
                    if s >= 1:
                        pl.semaphore_wait(xc, 1)
                    xd(s + 1).start()

            if s == 0:
                abuf[0] = p_ref[...]
            else:
                ad(s - 1).wait_send()
                if s <= 6:
                    credit(ac, up)
                ad(s - 1).wait_recv()
                abuf[s % 2] = abuf[s % 2] + p_ref[...]
                if s >= 2:
                    pl.semaphore_wait(ac, 1)
            ad(s).start()

    aa(7).wait_recv()
    ab(7).wait_recv()
    aa(7).wait_send()
    ab(7).wait_send()


def kernel(x, Wq, Wk, Wv, Wo):
    vm = pl.BlockSpec(memory_space=pltpu.VMEM)
    n_x, n_a = N_DEV - 1, N_DEV
    wq3 = Wq.reshape(D, HQ_LOC, DH).transpose(1, 0, 2)
    wk3 = Wk.reshape(D, HQ_LOC, DH).transpose(1, 0, 2)
    wv3 = Wv.reshape(D, HQ_LOC, DH).transpose(1, 0, 2)
    wo3 = Wo.reshape(HQ_LOC, DH, D)
    return pl.pallas_call(
        _body,
        out_shape=jax.ShapeDtypeStruct((B_LOC, SQ, D), jnp.float32),
        in_specs=[vm] * 8,
        out_specs=vm,
        scratch_shapes=[
            pltpu.VMEM((2, SQ, D), jnp.bfloat16),
            pltpu.VMEM((2, SQ, D), jnp.bfloat16),
            pltpu.VMEM((2, SQ, D), jnp.float32),
            pltpu.VMEM((2, SQ, D), jnp.float32),
            pltpu.VMEM((SQ, D), jnp.float32),
            pltpu.VMEM((2, SQ, D), jnp.bfloat16),
            pltpu.SemaphoreType.DMA((n_x,)),
            pltpu.SemaphoreType.DMA((n_x,)),
            pltpu.SemaphoreType.DMA((n_x,)),
            pltpu.SemaphoreType.DMA((n_x,)),
            pltpu.SemaphoreType.DMA((n_a,)),
            pltpu.SemaphoreType.DMA((n_a,)),
            pltpu.SemaphoreType.DMA((n_a,)),
            pltpu.SemaphoreType.DMA((n_a,)),
            pltpu.SemaphoreType.REGULAR,
            pltpu.SemaphoreType.REGULAR,
            pltpu.SemaphoreType.REGULAR,
            pltpu.SemaphoreType.REGULAR,
        ],
        compiler_params=pltpu.CompilerParams(
            collective_id=0, vmem_limit_bytes=62 * 1024 * 1024
        ),
    )(x, wq3, wk3, wv3, wo3,
      jnp.asarray(_COS_T), jnp.asarray(_SIN_A), jnp.asarray(_SIN_B))


# device time: 332606 ns/iter; 2.6993x vs baseline; 1.0353x over previous
import numpy as np
import jax
import jax.numpy as jnp
from jax import lax
from jax.experimental import pallas as pl
from jax.experimental.pallas import tpu as pltpu

N_DEV = 8
B, SQ, D = 16, 512, 1024
B_LOC = B // N_DEV
HQ_LOC, DH = 8, 128
SCALE = 0.08838834764831843

MESH = pl.DeviceIdType.MESH


def _rope_tables():
    inv = 1.0 / (10000.0 ** (np.arange(0, DH, 2) / DH))
    pos = np.arange(SQ)[:, None] * inv[None, :]
    cos = np.repeat(np.cos(pos), 2, axis=-1).astype(np.float32)
    sin = np.repeat(np.sin(pos), 2, axis=-1).astype(np.float32)
    even = (np.arange(DH) % 2 == 0)[None, :]
    sin_a = np.where(even, -sin, 0.0).astype(np.float32)
    sin_b = np.where(~even, sin, 0.0).astype(np.float32)
    return cos, sin_a, sin_b


_COS_T, _SIN_A, _SIN_B = _rope_tables()


def _body(x_ref, wq_ref, wk_ref, wv_ref, wo_ref, cos_ref, sa_ref, sb_ref,
          out_ref, xa_buf, xb_buf, aa_buf, ab_buf, p_ref, xsend, afin,
          xa_s, xa_r, xb_s, xb_r, aa_s, aa_r, ab_s, ab_r,
          xa_c, xb_c, aa_c, ab_c):
    my = lax.axis_index("i")
    left = lax.rem(my + N_DEV - 1, N_DEV)
    right = lax.rem(my + 1, N_DEV)

    sem = pltpu.get_barrier_semaphore()
    for nbr in (left, right):
        pl.semaphore_signal(sem, inc=1, device_id=(nbr,), device_id_type=MESH)
    pl.semaphore_wait(sem, 2)

    cos = cos_ref[...]
    sa = sa_ref[...]
    sb = sb_ref[...]

    def rope(t):
        return (t * cos
                + pltpu.roll(t, DH - 1, 1) * sa
                + pltpu.roll(t, 1, 1) * sb)

    def compute_partial(x):
        p_ref[...] = jnp.zeros((SQ, D), jnp.float32)

        def head(h, _):
            qh = rope(jnp.dot(x, wq_ref[h],
                              preferred_element_type=jnp.float32))
            kh = rope(jnp.dot(x, wk_ref[h],
                              preferred_element_type=jnp.float32))
            vh = jnp.dot(x, wv_ref[h],
                         preferred_element_type=jnp.float32)
            s_ = lax.dot_general(
                qh, kh, (((1,), (1,)), ((), ())),
                preferred_element_type=jnp.float32) * SCALE
            s_ = s_ - jnp.max(s_, axis=1, keepdims=True)
            e = jnp.exp(s_)
            w = e / jnp.sum(e, axis=1, keepdims=True)
            ch = jnp.dot(w, vh, preferred_element_type=jnp.float32)
            p_ref[...] = p_ref[...] + jnp.dot(
                ch, wo_ref[h], preferred_element_type=jnp.float32)
            return 0

        lax.fori_loop(0, HQ_LOC, head, 0)

    def x_desc(j, xbuf, xin, ssem, rsem, dst_dev):
        return pltpu.make_async_remote_copy(
            src_ref=xin if j == 0 else xbuf.at[j % 2],
            dst_ref=xbuf.at[(j + 1) % 2],
            send_sem=ssem.at[j], recv_sem=rsem.at[j],
            device_id=(dst_dev,), device_id_type=MESH)

    def a_desc(j, abuf, out_slot, ssem, rsem, dst_dev):
        return pltpu.make_async_remote_copy(
            src_ref=abuf.at[j % 2],
            dst_ref=afin.at[out_slot] if j == 7 else abuf.at[(j + 1) % 2],
            send_sem=ssem.at[j], recv_sem=rsem.at[j],
            device_id=(dst_dev,), device_id_type=MESH)

    xa = lambda j: x_desc(j, xa_buf, xsend.at[0], xa_s, xa_r, left)
    aa = lambda j: a_desc(j, aa_buf, 0, aa_s, aa_r, left)
    xb = lambda j: x_desc(j, xb_buf, xsend.at[1], xb_s, xb_r, right)
    ab = lambda j: a_desc(j, ab_buf, 1, ab_s, ab_r, right)

    def credit(sem, dev):
        pl.semaphore_signal(sem, inc=1, device_id=(dev,),
                            device_id_type=MESH)

    xsend[0] = x_ref[0].astype(jnp.bfloat16)
    xsend[1] = x_ref[1].astype(jnp.bfloat16)
    xa(0).start()
    xb(0).start()

    for s in range(N_DEV):
        for (x_in, xbuf, abuf, xd, ad, xc, ac, up) in (
            (x_ref, xa_buf, aa_buf, xa, aa, xa_c, aa_c, right),
            (x_ref, xb_buf, ab_buf, xb, ab, xb_c, ab_c, left),
        ):
            half = 0 if xbuf is xa_buf else 1
            x_cur = (x_in[half] if s == 0
                     else xbuf[s % 2].astype(jnp.float32))
            compute_partial(x_cur)

            if s <= 6:
                xd(s).wait_send()
                if 1 <= s <= 5:
                    credit(xc, up)
                xd(s).wait_recv()
                if s <= 5:
                    if s >= 1:
                        pl.semaphore_wait(xc, 1)
                    xd(s + 1).start()

            if s == 0:
                abuf[0] = p_ref[...].astype(jnp.bfloat16)
            else:
                ad(s - 1).wait_send()
                if s <= 6:
                    credit(ac, up)
                ad(s - 1).wait_recv()
                abuf[s % 2] = (abuf[s % 2].astype(jnp.float32)
                               + p_ref[...]).astype(jnp.bfloat16)
                if s >= 2:
                    pl.semaphore_wait(ac, 1)
            ad(s).start()

    aa(7).wait_recv()
    out_ref[0] = afin[0].astype(jnp.float32)
    ab(7).wait_recv()
    out_ref[1] = afin[1].astype(jnp.float32)
    aa(7).wait_send()
    ab(7).wait_send()


def kernel(x, Wq, Wk, Wv, Wo):
    vm = pl.BlockSpec(memory_space=pltpu.VMEM)
    n_x, n_a = N_DEV - 1, N_DEV
    wq3 = Wq.reshape(D, HQ_LOC, DH).transpose(1, 0, 2)
    wk3 = Wk.reshape(D, HQ_LOC, DH).transpose(1, 0, 2)
    wv3 = Wv.reshape(D, HQ_LOC, DH).transpose(1, 0, 2)
    wo3 = Wo.reshape(HQ_LOC, DH, D)
    return pl.pallas_call(
        _body,
        out_shape=jax.ShapeDtypeStruct((B_LOC, SQ, D), jnp.float32),
        in_specs=[vm] * 8,
        out_specs=vm,
        scratch_shapes=[
            pltpu.VMEM((2, SQ, D), jnp.bfloat16),
            pltpu.VMEM((2, SQ, D), jnp.bfloat16),
            pltpu.VMEM((2, SQ, D), jnp.bfloat16),
            pltpu.VMEM((2, SQ, D), jnp.bfloat16),
            pltpu.VMEM((SQ, D), jnp.float32),
            pltpu.VMEM((2, SQ, D), jnp.bfloat16),
            pltpu.VMEM((2, SQ, D), jnp.bfloat16),
            pltpu.SemaphoreType.DMA((n_x,)),
            pltpu.SemaphoreType.DMA((n_x,)),
            pltpu.SemaphoreType.DMA((n_x,)),
            pltpu.SemaphoreType.DMA((n_x,)),
            pltpu.SemaphoreType.DMA((n_a,)),
            pltpu.SemaphoreType.DMA((n_a,)),
            pltpu.SemaphoreType.DMA((n_a,)),
            pltpu.SemaphoreType.DMA((n_a,)),
            pltpu.SemaphoreType.REGULAR,
            pltpu.SemaphoreType.REGULAR,
            pltpu.SemaphoreType.REGULAR,
            pltpu.SemaphoreType.REGULAR,
        ],
        compiler_params=pltpu.CompilerParams(
            collective_id=0, vmem_limit_bytes=62 * 1024 * 1024
        ),
    )(x, wq3, wk3, wv3, wo3,
      jnp.asarray(_COS_T), jnp.asarray(_SIN_A), jnp.asarray(_SIN_B))


# device time: 325678 ns/iter; 2.7567x vs baseline; 1.0213x over previous
import numpy as np
import jax
import jax.numpy as jnp
from jax import lax
from jax.experimental import pallas as pl
from jax.experimental.pallas import tpu as pltpu

N_DEV = 8
B, SQ, D = 16, 512, 1024
B_LOC = B // N_DEV
HQ_LOC, DH = 8, 128
SCALE = 0.08838834764831843

MESH = pl.DeviceIdType.MESH


def _rope_tables():
    inv = 1.0 / (10000.0 ** (np.arange(0, DH, 2) / DH))
    pos = np.arange(SQ)[:, None] * inv[None, :]
    cos = np.repeat(np.cos(pos), 2, axis=-1).astype(np.float32)
    sin = np.repeat(np.sin(pos), 2, axis=-1).astype(np.float32)
    even = (np.arange(DH) % 2 == 0)[None, :]
    sin_a = np.where(even, -sin, 0.0).astype(np.float32)
    sin_b = np.where(~even, sin, 0.0).astype(np.float32)
    return cos, sin_a, sin_b


_COS_T, _SIN_A, _SIN_B = _rope_tables()


def _body(x_ref, wq_ref, wk_ref, wv_ref, wo_ref, cos_ref, sa_ref, sb_ref,
          out_ref, xa_buf, xb_buf, aa_buf, ab_buf, p_ref, xsend, afin,
          c3_ref,
          xa_s, xa_r, xb_s, xb_r, aa_s, aa_r, ab_s, ab_r,
          xa_c, xb_c, aa_c, ab_c):
    my = lax.axis_index("i")
    left = lax.rem(my + N_DEV - 1, N_DEV)
    right = lax.rem(my + 1, N_DEV)

    sem = pltpu.get_barrier_semaphore()
    for nbr in (left, right):
        pl.semaphore_signal(sem, inc=1, device_id=(nbr,), device_id_type=MESH)
    pl.semaphore_wait(sem, 2)

    cos = cos_ref[...]
    sa = sa_ref[...]
    sb = sb_ref[...]

    def rope(t):
        return (t * cos
                + pltpu.roll(t, DH - 1, 1) * sa
                + pltpu.roll(t, 1, 1) * sb)

    def compute_partial(x):
        def head(h, _):
            qh = rope(jnp.dot(x, wq_ref[h],
                              preferred_element_type=jnp.float32))
            kh = rope(jnp.dot(x, wk_ref[h],
                              preferred_element_type=jnp.float32))
            vh = jnp.dot(x, wv_ref[h],
                         preferred_element_type=jnp.float32)
            s_ = lax.dot_general(
                qh, kh, (((1,), (1,)), ((), ())),
                preferred_element_type=jnp.float32) * SCALE
            e = jnp.exp(s_)
            w = e / jnp.sum(e, axis=1, keepdims=True)
            c3_ref[h] = jnp.dot(w, vh, preferred_element_type=jnp.float32)
            return 0

        lax.fori_loop(0, HQ_LOC, head, 0)
        p = jnp.dot(c3_ref[0], wo_ref[0], preferred_element_type=jnp.float32)
        for h in range(1, HQ_LOC):
            p = p + jnp.dot(c3_ref[h], wo_ref[h],
                            preferred_element_type=jnp.float32)
        p_ref[...] = p

    def x_desc(j, xbuf, xin, ssem, rsem, dst_dev):
        return pltpu.make_async_remote_copy(
            src_ref=xin if j == 0 else xbuf.at[j % 2],
            dst_ref=xbuf.at[(j + 1) % 2],
            send_sem=ssem.at[j], recv_sem=rsem.at[j],
            device_id=(dst_dev,), device_id_type=MESH)

    def a_desc(j, abuf, out_slot, ssem, rsem, dst_dev):
        return pltpu.make_async_remote_copy(
            src_ref=abuf.at[j % 2],
            dst_ref=afin.at[out_slot] if j == 7 else abuf.at[(j + 1) % 2],
            send_sem=ssem.at[j], recv_sem=rsem.at[j],
            device_id=(dst_dev,), device_id_type=MESH)

    xa = lambda j: x_desc(j, xa_buf, xsend.at[0], xa_s, xa_r, left)
    aa = lambda j: a_desc(j, aa_buf, 0, aa_s, aa_r, left)
    xb = lambda j: x_desc(j, xb_buf, xsend.at[1], xb_s, xb_r, right)
    ab = lambda j: a_desc(j, ab_buf, 1, ab_s, ab_r, right)

    def credit(sem, dev):
        pl.semaphore_signal(sem, inc=1, device_id=(dev,),
                            device_id_type=MESH)

    xsend[0] = x_ref[0].astype(jnp.bfloat16)
    xsend[1] = x_ref[1].astype(jnp.bfloat16)
    xa(0).start()
    xb(0).start()

    for s in range(N_DEV):
        for (x_in, xbuf, abuf, xd, ad, xc, ac, up) in (
            (x_ref, xa_buf, aa_buf, xa, aa, xa_c, aa_c, right),
            (x_ref, xb_buf, ab_buf, xb, ab, xb_c, ab_c, left),
        ):
            half = 0 if xbuf is xa_buf else 1
            x_cur = (x_in[half] if s == 0
                     else xbuf[s % 2].astype(jnp.float32))
            compute_partial(x_cur)

            if s <= 6:
                xd(s).wait_send()
                if 1 <= s <= 5:
                    credit(xc, up)
                xd(s).wait_recv()
                if s <= 5:
                    if s >= 1:
                        pl.semaphore_wait(xc, 1)
                    xd(s + 1).start()

            if s == 0:
                abuf[0] = p_ref[...].astype(jnp.bfloat16)
            else:
                ad(s - 1).wait_send()
                if s <= 6:
                    credit(ac, up)
                ad(s - 1).wait_recv()
                abuf[s % 2] = (abuf[s % 2].astype(jnp.float32)
                               + p_ref[...]).astype(jnp.bfloat16)
                if s >= 2:
                    pl.semaphore_wait(ac, 1)
            ad(s).start()

    aa(7).wait_recv()
    out_ref[0] = afin[0].astype(jnp.float32)
    ab(7).wait_recv()
    out_ref[1] = afin[1].astype(jnp.float32)
    aa(7).wait_send()
    ab(7).wait_send()


def kernel(x, Wq, Wk, Wv, Wo):
    vm = pl.BlockSpec(memory_space=pltpu.VMEM)
    n_x, n_a = N_DEV - 1, N_DEV
    wq3 = Wq.reshape(D, HQ_LOC, DH).transpose(1, 0, 2)
    wk3 = Wk.reshape(D, HQ_LOC, DH).transpose(1, 0, 2)
    wv3 = Wv.reshape(D, HQ_LOC, DH).transpose(1, 0, 2)
    wo3 = Wo.reshape(HQ_LOC, DH, D)
    return pl.pallas_call(
        _body,
        out_shape=jax.ShapeDtypeStruct((B_LOC, SQ, D), jnp.float32),
        in_specs=[vm] * 8,
        out_specs=vm,
        scratch_shapes=[
            pltpu.VMEM((2, SQ, D), jnp.bfloat16),
            pltpu.VMEM((2, SQ, D), jnp.bfloat16),
            pltpu.VMEM((2, SQ, D), jnp.bfloat16),
            pltpu.VMEM((2, SQ, D), jnp.bfloat16),
            pltpu.VMEM((SQ, D), jnp.float32),
            pltpu.VMEM((2, SQ, D), jnp.bfloat16),
            pltpu.VMEM((2, SQ, D), jnp.bfloat16),
            pltpu.VMEM((HQ_LOC, SQ, DH), jnp.float32),
            pltpu.SemaphoreType.DMA((n_x,)),
            pltpu.SemaphoreType.DMA((n_x,)),
            pltpu.SemaphoreType.DMA((n_x,)),
            pltpu.SemaphoreType.DMA((n_x,)),
            pltpu.SemaphoreType.DMA((n_a,)),
            pltpu.SemaphoreType.DMA((n_a,)),
            pltpu.SemaphoreType.DMA((n_a,)),
            pltpu.SemaphoreType.DMA((n_a,)),
            pltpu.SemaphoreType.REGULAR,
            pltpu.SemaphoreType.REGULAR,
            pltpu.SemaphoreType.REGULAR,
            pltpu.SemaphoreType.REGULAR,
        ],
        compiler_params=pltpu.CompilerParams(
            collective_id=0, vmem_limit_bytes=62 * 1024 * 1024
        ),
    )(x, wq3, wk3, wv3, wo3,
      jnp.asarray(_COS_T), jnp.asarray(_SIN_A), jnp.asarray(_SIN_B))
